# initial kernel scaffold (unmeasured)
import jax
import jax.numpy as jnp
from jax import lax
from jax.experimental import pallas as pl
from jax.experimental.pallas import tpu as pltpu

N_DEV = 16
SQ = 1024
SKV = 1024
D = 1024
HQ = 8
DH = 128
BLK = 64
CHUNK = SQ // N_DEV
SCALE = 0.08838834764831843


def kernel(x, Wq, K_ext, V_ext, Wo):
    def body(x_ref, wq_hbm, k_ref, v_ref, wo_hbm, out_ref,
             wq_ref, wo_ref, partial_ref, rs_ref,
             load_sems, send_a, recv_a, send_b, recv_b):
        me = lax.axis_index("i")

        wq_cp = pltpu.make_async_copy(
            wq_hbm.at[:, pl.ds(me * D, D)], wq_ref, load_sems.at[0])
        wq_cp.start()
        wo_cp = pltpu.make_async_copy(
            wo_hbm.at[pl.ds(me * D, D), :], wo_ref, load_sems.at[1])
        wo_cp.start()

        barrier = pltpu.get_barrier_semaphore()
        for off in range(1, N_DEV):
            peer = lax.rem(me + off, N_DEV)
            pl.semaphore_signal(barrier, inc=1, device_id=(peer,),
                                device_id_type=pl.DeviceIdType.MESH)
        pl.semaphore_wait(barrier, N_DEV - 1)

        x_bf = x_ref[0].astype(jnp.bfloat16)
        wq_cp.wait()
        q = jnp.dot(x_bf, wq_ref[...].astype(jnp.bfloat16),
                    preferred_element_type=jnp.float32)
        q = q.astype(jnp.bfloat16)

        rowb = lax.broadcasted_iota(jnp.int32, (SQ, SKV), 0) // BLK
        colb = lax.broadcasted_iota(jnp.int32, (SQ, SKV), 1) // BLK
        mask = colb <= rowb

        ctx_parts = []
        for h in range(HQ):
            qh = q[:, h * DH:(h + 1) * DH]
            kh = k_ref[0, :, h, :].astype(jnp.bfloat16)
            s = lax.dot_general(qh, kh, (((1,), (1,)), ((), ())),
                                preferred_element_type=jnp.float32)
            s = s * SCALE
            s = jnp.where(mask, s, -1e9)
            m = jnp.max(s, axis=1, keepdims=True)
            w = jnp.exp(s - m)
            w = w / jnp.sum(w, axis=1, keepdims=True)
            vh = v_ref[0, :, h, :].astype(jnp.bfloat16)
            ctx_parts.append(
                jnp.dot(w.astype(jnp.bfloat16), vh,
                        preferred_element_type=jnp.float32))
        ctx = jnp.concatenate(ctx_parts, axis=1).astype(jnp.bfloat16)

        wo_cp.wait()
        partial_ref[...] = jnp.dot(
            ctx, wo_ref[...].astype(jnp.bfloat16),
            preferred_element_type=jnp.float32).astype(jnp.bfloat16)

        my_rows = pl.ds(me * CHUNK, CHUNK)
        loc = pltpu.make_async_copy(
            partial_ref.at[my_rows, :], rs_ref.at[me], load_sems.at[2])
        loc.start()
        sends_a = []
        for off in range(1, N_DEV):
            peer = lax.rem(me + off, N_DEV)
            rdma = pltpu.make_async_remote_copy(
                src_ref=partial_ref.at[pl.ds(peer * CHUNK, CHUNK), :],
                dst_ref=rs_ref.at[me],
                send_sem=send_a, recv_sem=recv_a,
                device_id=(peer,), device_id_type=pl.DeviceIdType.MESH)
            rdma.start()
            sends_a.append(rdma)
        loc.wait()
        for _ in range(N_DEV - 1):
            pltpu.make_async_remote_copy(
                src_ref=rs_ref.at[0], dst_ref=rs_ref.at[0],
                send_sem=send_a, recv_sem=recv_a,
                device_id=(me,), device_id_type=pl.DeviceIdType.MESH,
            ).wait_recv()
        for rdma in sends_a:
            rdma.wait_send()

        reduced = jnp.sum(rs_ref[...].astype(jnp.float32), axis=0)
        out_ref[0, my_rows, :] = reduced.astype(jnp.bfloat16)

        sends_b = []
        for off in range(1, N_DEV):
            peer = lax.rem(me + off, N_DEV)
            rdma = pltpu.make_async_remote_copy(
                src_ref=out_ref.at[0, my_rows, :],
                dst_ref=out_ref.at[0, my_rows, :],
                send_sem=send_b, recv_sem=recv_b,
                device_id=(peer,), device_id_type=pl.DeviceIdType.MESH)
            rdma.start()
            sends_b.append(rdma)
        for _ in range(N_DEV - 1):
            pltpu.make_async_remote_copy(
                src_ref=out_ref.at[0, pl.ds(0, CHUNK), :],
                dst_ref=out_ref.at[0, pl.ds(0, CHUNK), :],
                send_sem=send_b, recv_sem=recv_b,
                device_id=(me,), device_id_type=pl.DeviceIdType.MESH,
            ).wait_recv()
        for rdma in sends_b:
            rdma.wait_send()

    return pl.pallas_call(
        body,
        out_shape=jax.ShapeDtypeStruct((1, SQ, D), jnp.bfloat16),
        in_specs=[
            pl.BlockSpec(memory_space=pltpu.VMEM),
            pl.BlockSpec(memory_space=pltpu.ANY),
            pl.BlockSpec(memory_space=pltpu.VMEM),
            pl.BlockSpec(memory_space=pltpu.VMEM),
            pl.BlockSpec(memory_space=pltpu.ANY),
        ],
        out_specs=pl.BlockSpec(memory_space=pltpu.VMEM),
        scratch_shapes=[
            pltpu.VMEM((D, D), jnp.float32),
            pltpu.VMEM((D, D), jnp.float32),
            pltpu.VMEM((SQ, D), jnp.bfloat16),
            pltpu.VMEM((N_DEV, CHUNK, D), jnp.bfloat16),
            pltpu.SemaphoreType.DMA((3,)),
            pltpu.SemaphoreType.DMA,
            pltpu.SemaphoreType.DMA,
            pltpu.SemaphoreType.DMA,
            pltpu.SemaphoreType.DMA,
        ],
        compiler_params=pltpu.CompilerParams(collective_id=0),
    )(x, Wq, K_ext, V_ext, Wo)


# baseline (device time: 81252 ns/iter reference)
import jax
import jax.numpy as jnp
from jax import lax
from jax.experimental import pallas as pl
from jax.experimental.pallas import tpu as pltpu

N_DEV = 16
SQ = 1024
SKV = 1024
D = 1024
HQ = 8
DH = 128
BLK = 64
CHUNK = SQ // N_DEV
SCALE = 0.08838834764831843


def kernel(x, Wq, K_ext, V_ext, Wo):
    def body(x_ref, wq_hbm, k_ref, v_ref, wo_hbm, out_ref,
             wq_ref, wo_ref, partial_ref, rs_ref,
             load_sems, send_a, recv_a, send_b, recv_b):
        me = lax.axis_index("i")

        wq_cp = pltpu.make_async_copy(
            wq_hbm.at[:, pl.ds(me * D, D)], wq_ref, load_sems.at[0])
        wq_cp.start()
        wo_cp = pltpu.make_async_copy(
            wo_hbm.at[pl.ds(me * D, D), :], wo_ref, load_sems.at[1])
        wo_cp.start()

        barrier = pltpu.get_barrier_semaphore()
        for off in range(1, N_DEV):
            peer = lax.rem(me + off, N_DEV)
            pl.semaphore_signal(barrier, inc=1, device_id=(peer,),
                                device_id_type=pl.DeviceIdType.MESH)
        pl.semaphore_wait(barrier, N_DEV - 1)

        x_bf = x_ref[0].astype(jnp.bfloat16)
        wq_cp.wait()
        q = jnp.dot(x_bf, wq_ref[...].astype(jnp.bfloat16),
                    preferred_element_type=jnp.float32)
        q = q.astype(jnp.bfloat16)

        rowb = lax.broadcasted_iota(jnp.int32, (SQ, SKV), 0) // BLK
        colb = lax.broadcasted_iota(jnp.int32, (SQ, SKV), 1) // BLK
        mask = colb <= rowb

        ctx_parts = []
        for h in range(HQ):
            qh = q[:, h * DH:(h + 1) * DH]
            kh = k_ref[0, :, h, :].astype(jnp.bfloat16)
            s = lax.dot_general(qh, kh, (((1,), (1,)), ((), ())),
                                preferred_element_type=jnp.float32)
            s = s * SCALE
            s = jnp.where(mask, s, -1e9)
            m = jnp.max(s, axis=1, keepdims=True)
            w = jnp.exp(s - m)
            w = w / jnp.sum(w, axis=1, keepdims=True)
            vh = v_ref[0, :, h, :].astype(jnp.bfloat16)
            ctx_parts.append(
                jnp.dot(w.astype(jnp.bfloat16), vh,
                        preferred_element_type=jnp.float32))
        ctx = jnp.concatenate(ctx_parts, axis=1).astype(jnp.bfloat16)

        wo_cp.wait()
        partial_ref[...] = jnp.dot(
            ctx, wo_ref[...].astype(jnp.bfloat16),
            preferred_element_type=jnp.float32).astype(jnp.bfloat16)

        my_rows = pl.ds(me * CHUNK, CHUNK)
        loc = pltpu.make_async_copy(
            partial_ref.at[my_rows, :], rs_ref.at[me], load_sems.at[2])
        loc.start()
        sends_a = []
        for off in range(1, N_DEV):
            peer = lax.rem(me + off, N_DEV)
            rdma = pltpu.make_async_remote_copy(
                src_ref=partial_ref.at[pl.ds(peer * CHUNK, CHUNK), :],
                dst_ref=rs_ref.at[me],
                send_sem=send_a, recv_sem=recv_a,
                device_id=(peer,), device_id_type=pl.DeviceIdType.MESH)
            rdma.start()
            sends_a.append(rdma)
        loc.wait()
        for _ in range(N_DEV - 1):
            pltpu.make_async_remote_copy(
                src_ref=rs_ref.at[0], dst_ref=rs_ref.at[0],
                send_sem=send_a, recv_sem=recv_a,
                device_id=(me,), device_id_type=pl.DeviceIdType.MESH,
            ).wait_recv()
        for rdma in sends_a:
            rdma.wait_send()

        reduced = jnp.sum(rs_ref[...].astype(jnp.float32), axis=0)
        out_ref[0, my_rows, :] = reduced.astype(jnp.bfloat16)

        sends_b = []
        for off in range(1, N_DEV):
            peer = lax.rem(me + off, N_DEV)
            rdma = pltpu.make_async_remote_copy(
                src_ref=out_ref.at[0, my_rows, :],
                dst_ref=out_ref.at[0, my_rows, :],
                send_sem=send_b, recv_sem=recv_b,
                device_id=(peer,), device_id_type=pl.DeviceIdType.MESH)
            rdma.start()
            sends_b.append(rdma)
        for _ in range(N_DEV - 1):
            pltpu.make_async_remote_copy(
                src_ref=out_ref.at[0, pl.ds(0, CHUNK), :],
                dst_ref=out_ref.at[0, pl.ds(0, CHUNK), :],
                send_sem=send_b, recv_sem=recv_b,
                device_id=(me,), device_id_type=pl.DeviceIdType.MESH,
            ).wait_recv()
        for rdma in sends_b:
            rdma.wait_send()

    return pl.pallas_call(
        body,
        out_shape=jax.ShapeDtypeStruct((1, SQ, D), jnp.bfloat16),
        in_specs=[
            pl.BlockSpec(memory_space=pltpu.VMEM),
            pl.BlockSpec(memory_space=pltpu.MemorySpace.HBM),
            pl.BlockSpec(memory_space=pltpu.VMEM),
            pl.BlockSpec(memory_space=pltpu.VMEM),
            pl.BlockSpec(memory_space=pltpu.MemorySpace.HBM),
        ],
        out_specs=pl.BlockSpec(memory_space=pltpu.VMEM),
        scratch_shapes=[
            pltpu.VMEM((D, D), jnp.float32),
            pltpu.VMEM((D, D), jnp.float32),
            pltpu.VMEM((SQ, D), jnp.bfloat16),
            pltpu.VMEM((N_DEV, CHUNK, D), jnp.bfloat16),
            pltpu.SemaphoreType.DMA((3,)),
            pltpu.SemaphoreType.DMA,
            pltpu.SemaphoreType.DMA,
            pltpu.SemaphoreType.DMA,
            pltpu.SemaphoreType.DMA,
        ],
        compiler_params=pltpu.CompilerParams(collective_id=0),
    )(x, Wq, K_ext, V_ext, Wo)
